# baseline (speedup 1.0000x reference)
import functools

import jax
import jax.numpy as jnp
from jax import lax
from jax.experimental import pallas as pl
from jax.experimental.pallas import tpu as pltpu
from jax.experimental.pallas import tpu_sc as plsc

_EMB_DIM = 64
_BATCH = 16384
_N_NEG = 5
_NW = 32
_CHUNK = 128
_BLOCKS = 4
_BPW = _BATCH // _NW


def _log_sigmoid_vec(x):
    e = jnp.exp(-jnp.abs(x))
    t = e / (2.0 + e)
    t2 = t * t
    log1pe = 2.0 * t * (1.0 + t2 * (1.0 / 3.0 + t2 * (0.2 + t2 * (1.0 / 7.0 + t2 * (1.0 / 9.0)))))
    return jnp.minimum(x, 0.0) - log1pe


def _sc_loss_partials(table, iu, iv, ineg):
    mesh = plsc.VectorSubcoreMesh(core_axis_name="c", subcore_axis_name="s")
    info = plsc.get_sparse_core_info()
    nc = info.num_cores

    @functools.partial(
        pl.kernel,
        mesh=mesh,
        out_type=jax.ShapeDtypeStruct((_NW, 16), jnp.float32),
        scratch_types=[
            pltpu.VMEM((_BLOCKS, _CHUNK), jnp.int32),
            pltpu.VMEM((_BLOCKS, _CHUNK), jnp.int32),
            pltpu.VMEM((_BLOCKS * _N_NEG, _CHUNK), jnp.int32),
            pltpu.VMEM((_CHUNK, _EMB_DIM), jnp.float32),
            pltpu.VMEM((_CHUNK, _EMB_DIM), jnp.float32),
            pltpu.VMEM((_CHUNK, _EMB_DIM), jnp.float32),
            pltpu.VMEM((_CHUNK, _EMB_DIM), jnp.float32),
            pltpu.VMEM((_CHUNK * _N_NEG, _EMB_DIM), jnp.float32),
            pltpu.VMEM((_CHUNK * _N_NEG, _EMB_DIM), jnp.float32),
            pltpu.VMEM((16,), jnp.float32),
            pltpu.SemaphoreType.DMA,
        ],
        compiler_params=pltpu.CompilerParams(use_tc_tiling_on_sc=False,
                                             needs_layout_passes=False),
    )
    def k(table_hbm, iu_hbm, iv_hbm, ineg_hbm, out,
          iu_v, iv_v, ineg_v, uA, uB, vA, vB, nA, nB, acc_v, sem):
        wid = lax.axis_index("s") * nc + lax.axis_index("c")
        pltpu.sync_copy(iu_hbm.at[wid], iu_v)
        pltpu.sync_copy(iv_hbm.at[wid], iv_v)
        pltpu.sync_copy(ineg_hbm.at[wid], ineg_v)

        def fire(t, ub, vb, nb):
            cps = [
                pltpu.async_copy(table_hbm.at[iu_v.at[t]], ub, sem),
                pltpu.async_copy(table_hbm.at[iv_v.at[t]], vb, sem),
            ]
            for n5 in range(_N_NEG):
                cps.append(pltpu.async_copy(
                    table_hbm.at[ineg_v.at[_N_NEG * t + n5]],
                    nb.at[pl.ds(_CHUNK * n5, _CHUNK)], sem))
            return cps

        iota16 = lax.iota(jnp.int32, 16)

        def compute_block(ub, vb, nb, accs):
            def g_body(g, accs):
                acc_p, acc_n = accs
                e = g * 16 + iota16
                e5 = e * _N_NEG

                zero = jnp.zeros((16,), jnp.float32)

                @plsc.parallel_loop(0, _EMB_DIM, unroll=8,
                                    carry=(zero, zero, zero, zero, zero, zero))
                def dots(d, carry):
                    pos, nd0, nd1, nd2, nd3, nd4 = carry
                    nds = [nd0, nd1, nd2, nd3, nd4]
                    cols = jnp.full((16,), d, jnp.int32)
                    uvec = plsc.load_gather(ub, [e, cols])
                    vvec = plsc.load_gather(vb, [e, cols])
                    pos = pos + uvec * vvec
                    for n in range(_N_NEG):
                        nvec = plsc.load_gather(nb, [e5 + n, cols])
                        nds[n] = nds[n] + nvec * uvec
                    return (pos, nds[0], nds[1], nds[2], nds[3], nds[4])
                acc_p = acc_p + _log_sigmoid_vec(dots[0])
                for n in range(_N_NEG):
                    acc_n = acc_n + _log_sigmoid_vec(-dots[1 + n])
                return (acc_p, acc_n)

            return lax.fori_loop(0, _CHUNK // 16, g_body, accs)

        zero = jnp.zeros((16,), jnp.float32)
        accs = (zero, zero)
        cps = fire(0, uA, vA, nA)
        for t in range(_BLOCKS):
            for cp in cps:
                cp.wait()
            cur = (uA, vA, nA) if t % 2 == 0 else (uB, vB, nB)
            if t + 1 < _BLOCKS:
                nxt = (uB, vB, nB) if t % 2 == 0 else (uA, vA, nA)
                cps = fire(t + 1, *nxt)
            accs = compute_block(*cur, accs)

        acc = accs[0] * (1.0 / _BATCH) + accs[1] * (1.0 / (_BATCH * _N_NEG))
        acc_v[...] = acc
        pltpu.sync_copy(acc_v, out.at[wid])

    return k(table, iu, iv, ineg)


def kernel(pos_u, pos_v, neg_v, u_embeddings):
    iu = pos_u.reshape(_NW, _BLOCKS, _CHUNK)
    iv = pos_v.reshape(_NW, _BLOCKS, _CHUNK)
    ineg = neg_v.reshape(_NW, _BLOCKS * _N_NEG, _CHUNK)
    partials = _sc_loss_partials(u_embeddings, iu, iv, ineg)
    return -jnp.sum(partials)


# speedup vs baseline: 1.0020x; 1.0020x over previous
import functools

import jax
import jax.numpy as jnp
from jax import lax
from jax.experimental import pallas as pl
from jax.experimental.pallas import tpu as pltpu
from jax.experimental.pallas import tpu_sc as plsc

_EMB_DIM = 64
_BATCH = 16384
_N_NEG = 5
_NW = 32
_CHUNK = 128
_BLOCKS = 4
_BPW = _BATCH // _NW


def _log_sigmoid_vec(x):
    e = jnp.exp(-jnp.abs(x))
    t = e / (2.0 + e)
    t2 = t * t
    log1pe = 2.0 * t * (1.0 + t2 * (1.0 / 3.0 + t2 * (0.2 + t2 * (1.0 / 7.0 + t2 * (1.0 / 9.0)))))
    return jnp.minimum(x, 0.0) - log1pe


def _sc_loss_partials(table, iu, iv, ineg):
    mesh = plsc.VectorSubcoreMesh(core_axis_name="c", subcore_axis_name="s")
    info = plsc.get_sparse_core_info()
    nc = info.num_cores

    @functools.partial(
        pl.kernel,
        mesh=mesh,
        out_type=jax.ShapeDtypeStruct((_NW, 16), jnp.float32),
        scratch_types=[
            pltpu.VMEM((_BLOCKS, _CHUNK), jnp.int32),
            pltpu.VMEM((_BLOCKS, _CHUNK), jnp.int32),
            pltpu.VMEM((_BLOCKS * _N_NEG, _CHUNK), jnp.int32),
            pltpu.VMEM((_CHUNK, _EMB_DIM), jnp.float32),
            pltpu.VMEM((_CHUNK, _EMB_DIM), jnp.float32),
            pltpu.VMEM((_CHUNK, _EMB_DIM), jnp.float32),
            pltpu.VMEM((_CHUNK, _EMB_DIM), jnp.float32),
            pltpu.VMEM((_CHUNK * _N_NEG, _EMB_DIM), jnp.float32),
            pltpu.VMEM((_CHUNK * _N_NEG, _EMB_DIM), jnp.float32),
            pltpu.VMEM((16,), jnp.float32),
            pltpu.SemaphoreType.DMA,
        ],
        compiler_params=pltpu.CompilerParams(use_tc_tiling_on_sc=False,
                                             needs_layout_passes=False,
                                             disable_bounds_checks=True),
    )
    def k(table_hbm, iu_hbm, iv_hbm, ineg_hbm, out,
          iu_v, iv_v, ineg_v, uA, uB, vA, vB, nA, nB, acc_v, sem):
        wid = lax.axis_index("s") * nc + lax.axis_index("c")
        pltpu.sync_copy(iu_hbm.at[wid], iu_v)
        pltpu.sync_copy(iv_hbm.at[wid], iv_v)
        pltpu.sync_copy(ineg_hbm.at[wid], ineg_v)

        def fire(t, ub, vb, nb):
            cps = [
                pltpu.async_copy(table_hbm.at[iu_v.at[t]], ub, sem),
                pltpu.async_copy(table_hbm.at[iv_v.at[t]], vb, sem),
            ]
            for n5 in range(_N_NEG):
                cps.append(pltpu.async_copy(
                    table_hbm.at[ineg_v.at[_N_NEG * t + n5]],
                    nb.at[pl.ds(_CHUNK * n5, _CHUNK)], sem))
            return cps

        iota16 = lax.iota(jnp.int32, 16)

        def compute_block(ub, vb, nb, accs):
            def g_body(g, accs):
                acc_p, acc_n = accs
                e = g * 16 + iota16
                e5 = e * _N_NEG

                zero = jnp.zeros((16,), jnp.float32)

                @plsc.parallel_loop(0, _EMB_DIM, unroll=8,
                                    carry=(zero, zero, zero, zero, zero, zero))
                def dots(d, carry):
                    pos, nd0, nd1, nd2, nd3, nd4 = carry
                    nds = [nd0, nd1, nd2, nd3, nd4]
                    cols = jnp.full((16,), d, jnp.int32)
                    uvec = plsc.load_gather(ub, [e, cols])
                    vvec = plsc.load_gather(vb, [e, cols])
                    pos = pos + uvec * vvec
                    for n in range(_N_NEG):
                        nvec = plsc.load_gather(nb, [e5 + n, cols])
                        nds[n] = nds[n] + nvec * uvec
                    return (pos, nds[0], nds[1], nds[2], nds[3], nds[4])
                acc_p = acc_p + _log_sigmoid_vec(dots[0])
                for n in range(_N_NEG):
                    acc_n = acc_n + _log_sigmoid_vec(-dots[1 + n])
                return (acc_p, acc_n)

            return lax.fori_loop(0, _CHUNK // 16, g_body, accs)

        zero = jnp.zeros((16,), jnp.float32)
        accs = (zero, zero)
        cps = fire(0, uA, vA, nA)
        for t in range(_BLOCKS):
            for cp in cps:
                cp.wait()
            cur = (uA, vA, nA) if t % 2 == 0 else (uB, vB, nB)
            if t + 1 < _BLOCKS:
                nxt = (uB, vB, nB) if t % 2 == 0 else (uA, vA, nA)
                cps = fire(t + 1, *nxt)
            accs = compute_block(*cur, accs)

        acc = accs[0] * (1.0 / _BATCH) + accs[1] * (1.0 / (_BATCH * _N_NEG))
        acc_v[...] = acc
        pltpu.sync_copy(acc_v, out.at[wid])

    return k(table, iu, iv, ineg)


def kernel(pos_u, pos_v, neg_v, u_embeddings):
    iu = pos_u.reshape(_NW, _BLOCKS, _CHUNK)
    iv = pos_v.reshape(_NW, _BLOCKS, _CHUNK)
    ineg = neg_v.reshape(_NW, _BLOCKS * _N_NEG, _CHUNK)
    partials = _sc_loss_partials(u_embeddings, iu, iv, ineg)
    return -jnp.sum(partials)


# speedup vs baseline: 1.1562x; 1.1539x over previous
import functools

import jax
import jax.numpy as jnp
from jax import lax
from jax.experimental import pallas as pl
from jax.experimental.pallas import tpu as pltpu
from jax.experimental.pallas import tpu_sc as plsc

_EMB_DIM = 64
_BATCH = 16384
_N_NEG = 5
_NW = 32
_CHUNK = 128
_BLOCKS = 4
_BPW = _BATCH // _NW


def _log_sigmoid_vec(x):
    e = jnp.exp(-jnp.abs(x))
    t = e / (2.0 + e)
    t2 = t * t
    log1pe = 2.0 * t * (1.0 + t2 * (1.0 / 3.0 + t2 * (0.2 + t2 * (1.0 / 7.0 + t2 * (1.0 / 9.0)))))
    return jnp.minimum(x, 0.0) - log1pe


def _sc_loss_partials(table, iu, iv, ineg):
    mesh = plsc.VectorSubcoreMesh(core_axis_name="c", subcore_axis_name="s")
    info = plsc.get_sparse_core_info()
    nc = info.num_cores

    @functools.partial(
        pl.kernel,
        mesh=mesh,
        out_type=jax.ShapeDtypeStruct((_NW, 16), jnp.float32),
        scratch_types=[
            pltpu.VMEM((_BLOCKS, _CHUNK), jnp.int32),
            pltpu.VMEM((_BLOCKS, _CHUNK), jnp.int32),
            pltpu.VMEM((_BLOCKS * _N_NEG, _CHUNK), jnp.int32),
            pltpu.VMEM((_CHUNK, _EMB_DIM), jnp.float32),
            pltpu.VMEM((_CHUNK, _EMB_DIM), jnp.float32),
            pltpu.VMEM((_CHUNK, _EMB_DIM), jnp.float32),
            pltpu.VMEM((_CHUNK, _EMB_DIM), jnp.float32),
            pltpu.VMEM((_CHUNK * _N_NEG, _EMB_DIM), jnp.float32),
            pltpu.VMEM((_CHUNK * _N_NEG, _EMB_DIM), jnp.float32),
            pltpu.VMEM((16,), jnp.float32),
            pltpu.SemaphoreType.DMA,
        ],
        compiler_params=pltpu.CompilerParams(use_tc_tiling_on_sc=False,
                                             needs_layout_passes=False,
                                             disable_bounds_checks=True),
    )
    def k(table_hbm, iu_hbm, iv_hbm, ineg_hbm, out,
          iu_v, iv_v, ineg_v, uA, uB, vA, vB, nA, nB, acc_v, sem):
        wid = lax.axis_index("s") * nc + lax.axis_index("c")
        pltpu.sync_copy(iu_hbm.at[wid], iu_v)
        pltpu.sync_copy(iv_hbm.at[wid], iv_v)
        pltpu.sync_copy(ineg_hbm.at[wid], ineg_v)

        def fire(t, ub, vb, nb):
            cps = [
                pltpu.async_copy(table_hbm.at[iu_v.at[t]], ub, sem),
                pltpu.async_copy(table_hbm.at[iv_v.at[t]], vb, sem),
            ]
            for n5 in range(_N_NEG):
                cps.append(pltpu.async_copy(
                    table_hbm.at[ineg_v.at[_N_NEG * t + n5]],
                    nb.at[pl.ds(_CHUNK * n5, _CHUNK)], sem))
            return cps

        iota16 = lax.iota(jnp.int32, 16)

        def compute_block(ub, vb, nb, accs):
            def g_body(g, accs):
                acc_p, acc_n = accs
                e = g * 16 + iota16
                e5 = e * _N_NEG

                zero = jnp.zeros((16,), jnp.float32)

                @plsc.parallel_loop(0, _EMB_DIM, unroll=8,
                                    carry=(zero, zero, zero, zero, zero, zero))
                def dots(d, carry):
                    pos, nd0, nd1, nd2, nd3, nd4 = carry
                    nds = [nd0, nd1, nd2, nd3, nd4]
                    cols = (d + iota16) & (_EMB_DIM - 1)
                    uvec = plsc.load_gather(ub, [e, cols])
                    vvec = plsc.load_gather(vb, [e, cols])
                    pos = pos + uvec * vvec
                    for n in range(_N_NEG):
                        nvec = plsc.load_gather(nb, [e5 + n, cols])
                        nds[n] = nds[n] + nvec * uvec
                    return (pos, nds[0], nds[1], nds[2], nds[3], nds[4])
                acc_p = acc_p + _log_sigmoid_vec(dots[0])
                for n in range(_N_NEG):
                    acc_n = acc_n + _log_sigmoid_vec(-dots[1 + n])
                return (acc_p, acc_n)

            return lax.fori_loop(0, _CHUNK // 16, g_body, accs)

        zero = jnp.zeros((16,), jnp.float32)
        accs = (zero, zero)
        cps = fire(0, uA, vA, nA)
        for t in range(_BLOCKS):
            for cp in cps:
                cp.wait()
            cur = (uA, vA, nA) if t % 2 == 0 else (uB, vB, nB)
            if t + 1 < _BLOCKS:
                nxt = (uB, vB, nB) if t % 2 == 0 else (uA, vA, nA)
                cps = fire(t + 1, *nxt)
            accs = compute_block(*cur, accs)

        acc = accs[0] * (1.0 / _BATCH) + accs[1] * (1.0 / (_BATCH * _N_NEG))
        acc_v[...] = acc
        pltpu.sync_copy(acc_v, out.at[wid])

    return k(table, iu, iv, ineg)


def kernel(pos_u, pos_v, neg_v, u_embeddings):
    iu = pos_u.reshape(_NW, _BLOCKS, _CHUNK)
    iv = pos_v.reshape(_NW, _BLOCKS, _CHUNK)
    ineg = neg_v.reshape(_NW, _BLOCKS * _N_NEG, _CHUNK)
    partials = _sc_loss_partials(u_embeddings, iu, iv, ineg)
    return -jnp.sum(partials)
